# baseline (speedup 1.0000x reference)
import functools

import jax
import jax.numpy as jnp
from jax import lax
from jax.experimental import pallas as pl
from jax.experimental.pallas import tpu as pltpu
from jax.experimental.pallas import tpu_sc as plsc

B = 16384
D = 32
NC = 2
NS = 16
NW = NC * NS
BPW = B // NW
CHUNK = 128
NCH = BPW // CHUNK

_mesh = plsc.VectorSubcoreMesh(core_axis_name="c", subcore_axis_name="s")


@functools.partial(
    pl.kernel,
    mesh=_mesh,
    out_type=jax.ShapeDtypeStruct((B,), jnp.float32),
    scratch_types=[
        pltpu.VMEM((NCH, CHUNK), jnp.int32),
        pltpu.VMEM((NCH, CHUNK), jnp.int32),
        pltpu.VMEM((BPW, D), jnp.float32),
        pltpu.VMEM((BPW, D), jnp.float32),
        pltpu.VMEM((BPW,), jnp.float32),
        pltpu.VMEM((16, 521), jnp.float32),
        pltpu.SemaphoreType.DMA,
    ],
    compiler_params=pltpu.CompilerParams(use_tc_tiling_on_sc=False,
                                         needs_layout_passes=False),
)
def _mf_kernel(uids_hbm, iids_hbm, umem_hbm, imem_hbm, out_hbm,
               uidx_v, iidx_v, urows_v, irows_v, out_v, qT_v, sem):
    wid = lax.axis_index("s") * NC + lax.axis_index("c")
    base = wid * BPW

    for j in range(NCH):
        pltpu.sync_copy(uids_hbm.at[pl.ds(base + j * CHUNK, CHUNK)],
                        uidx_v.at[j])
        pltpu.sync_copy(iids_hbm.at[pl.ds(base + j * CHUNK, CHUNK)],
                        iidx_v.at[j])

    copies = []
    for j in range(NCH):
        copies.append(pltpu.async_copy(
            umem_hbm.at[uidx_v.at[j]],
            urows_v.at[pl.ds(j * CHUNK, CHUNK)], sem))
        copies.append(pltpu.async_copy(
            imem_hbm.at[iidx_v.at[j]],
            irows_v.at[pl.ds(j * CHUNK, CHUNK)], sem))
    for cp in copies:
        cp.wait()

    lane = lax.iota(jnp.int32, 16)

    def row_body(r, _):
        u0 = urows_v[r, pl.ds(0, 16)]
        u1 = urows_v[r, pl.ds(16, 16)]
        i0 = irows_v[r, pl.ds(0, 16)]
        i1 = irows_v[r, pl.ds(16, 16)]
        v = u0 * i0 + u1 * i1
        plsc.store_scatter(qT_v, [lane, jnp.full((16,), r, jnp.int32)], v)
        return 0

    lax.fori_loop(0, BPW, row_body, 0, unroll=8)

    def group_body(g, _):
        acc = qT_v[0, pl.ds(g * 16, 16)]
        for c in range(1, 16):
            acc = acc + qT_v[c, pl.ds(g * 16, 16)]
        out_v[pl.ds(g * 16, 16)] = acc
        return 0

    lax.fori_loop(0, BPW // 16, group_body, 0, unroll=2)

    pltpu.sync_copy(out_v, out_hbm.at[pl.ds(base, BPW)])


def kernel(userids, itemids, user_memory, item_memory):
    return _mf_kernel(userids.astype(jnp.int32), itemids.astype(jnp.int32),
                      user_memory, item_memory)


# speedup vs baseline: 1.4553x; 1.4553x over previous
import functools

import jax
import jax.numpy as jnp
from jax import lax
from jax.experimental import pallas as pl
from jax.experimental.pallas import tpu as pltpu
from jax.experimental.pallas import tpu_sc as plsc

B = 16384
D = 32
NC = 2
NS = 16
NW = NC * NS
BPW = B // NW
CHUNK = 256
NCHUNK = BPW // CHUNK
QSTRIDE = 521

_mesh = plsc.VectorSubcoreMesh(core_axis_name="c", subcore_axis_name="s")


@functools.partial(
    pl.kernel,
    mesh=_mesh,
    out_type=jax.ShapeDtypeStruct((B,), jnp.float32),
    scratch_types=[
        pltpu.VMEM((BPW,), jnp.int32),
        pltpu.VMEM((BPW,), jnp.int32),
        pltpu.VMEM((CHUNK, D), jnp.float32),
        pltpu.VMEM((CHUNK, D), jnp.float32),
        pltpu.VMEM((BPW,), jnp.float32),
        pltpu.VMEM((16 * QSTRIDE,), jnp.float32),
        pltpu.SemaphoreType.DMA,
    ],
    compiler_params=pltpu.CompilerParams(needs_layout_passes=False),
)
def _mf_kernel(uids_hbm, iids_hbm, umem_hbm, imem_hbm, out_hbm,
               uidx_v, iidx_v, urows_v, irows_v, out_v, qT_v, sem):
    wid = lax.axis_index("s") * NC + lax.axis_index("c")
    base = wid * BPW

    pltpu.sync_copy(uids_hbm.at[pl.ds(base, BPW)], uidx_v)
    pltpu.sync_copy(iids_hbm.at[pl.ds(base, BPW)], iidx_v)

    lane = lax.iota(jnp.int32, 16)
    qidx0 = lane * QSTRIDE

    for ch in range(NCHUNK):
        def enqueue_body(g, _, ch=ch):
            uv = uidx_v[pl.ds(ch * CHUNK + g * 16, 16)]
            iv = iidx_v[pl.ds(ch * CHUNK + g * 16, 16)]
            copies = []
            for j in range(16):
                copies.append(pltpu.async_copy(
                    umem_hbm.at[uv[j]], urows_v.at[g * 16 + j], sem))
                copies.append(pltpu.async_copy(
                    imem_hbm.at[iv[j]], irows_v.at[g * 16 + j], sem))
            for cp in copies:
                cp.wait()
            return 0

        lax.fori_loop(0, CHUNK // 16, enqueue_body, 0, unroll=1)

        def row_body(r, _, ch=ch):
            u0 = urows_v[r, pl.ds(0, 16)]
            u1 = urows_v[r, pl.ds(16, 16)]
            i0 = irows_v[r, pl.ds(0, 16)]
            i1 = irows_v[r, pl.ds(16, 16)]
            v = u0 * i0 + u1 * i1
            plsc.store_scatter(qT_v, [qidx0 + (ch * CHUNK + r)], v)
            return 0

        lax.fori_loop(0, CHUNK, row_body, 0, unroll=8)

    def group_body(g, _):
        acc = qT_v[pl.ds(g * 16, 16)]
        for c in range(1, 16):
            acc = acc + qT_v[pl.ds(c * QSTRIDE + g * 16, 16)]
        out_v[pl.ds(g * 16, 16)] = acc
        return 0

    lax.fori_loop(0, BPW // 16, group_body, 0, unroll=2)

    pltpu.sync_copy(out_v, out_hbm.at[pl.ds(base, BPW)])


def kernel(userids, itemids, user_memory, item_memory):
    return _mf_kernel(userids.astype(jnp.int32), itemids.astype(jnp.int32),
                      user_memory, item_memory)


# speedup vs baseline: 1.4745x; 1.0132x over previous
import functools

import jax
import jax.numpy as jnp
from jax import lax
from jax.experimental import pallas as pl
from jax.experimental.pallas import tpu as pltpu
from jax.experimental.pallas import tpu_sc as plsc

B = 16384
D = 32
NC = 2
NS = 16
NW = NC * NS
BPW = B // NW
CHUNK = 256
NCHUNK = BPW // CHUNK
QSTRIDE = 521

_mesh = plsc.VectorSubcoreMesh(core_axis_name="c", subcore_axis_name="s")


@functools.partial(
    pl.kernel,
    mesh=_mesh,
    out_type=jax.ShapeDtypeStruct((B,), jnp.float32),
    scratch_types=[
        pltpu.VMEM((BPW,), jnp.int32),
        pltpu.VMEM((BPW,), jnp.int32),
        pltpu.VMEM((CHUNK, D), jnp.float32),
        pltpu.VMEM((CHUNK, D), jnp.float32),
        pltpu.VMEM((BPW,), jnp.float32),
        pltpu.VMEM((16 * QSTRIDE,), jnp.float32),
        pltpu.SemaphoreType.DMA,
    ],
    compiler_params=pltpu.CompilerParams(needs_layout_passes=False),
)
def _mf_kernel(uids_hbm, iids_hbm, umem_hbm, imem_hbm, out_hbm,
               uidx_v, iidx_v, urows_v, irows_v, out_v, qT_v, sem):
    wid = lax.axis_index("s") * NC + lax.axis_index("c")
    base = wid * BPW

    pltpu.sync_copy(uids_hbm.at[pl.ds(base, BPW)], uidx_v)
    pltpu.sync_copy(iids_hbm.at[pl.ds(base, BPW)], iidx_v)

    lane = lax.iota(jnp.int32, 16)
    qidx0 = lane * QSTRIDE

    for ch in range(NCHUNK):
        def enqueue_body(h, _, ch=ch):
            def fire(g):
                uv = uidx_v[pl.ds(ch * CHUNK + g * 16, 16)]
                iv = iidx_v[pl.ds(ch * CHUNK + g * 16, 16)]
                copies = []
                for j in range(16):
                    copies.append(pltpu.async_copy(
                        umem_hbm.at[uv[j]], urows_v.at[g * 16 + j], sem))
                    copies.append(pltpu.async_copy(
                        imem_hbm.at[iv[j]], irows_v.at[g * 16 + j], sem))
                return copies

            c0 = fire(h * 2)
            c1 = fire(h * 2 + 1)
            for cp in c0 + c1:
                cp.wait()
            return 0

        lax.fori_loop(0, CHUNK // 32, enqueue_body, 0, unroll=1)

        def row_body(r, _, ch=ch):
            u0 = urows_v[r, pl.ds(0, 16)]
            u1 = urows_v[r, pl.ds(16, 16)]
            i0 = irows_v[r, pl.ds(0, 16)]
            i1 = irows_v[r, pl.ds(16, 16)]
            v = u0 * i0 + u1 * i1
            plsc.store_scatter(qT_v, [qidx0 + (ch * CHUNK + r)], v)
            return 0

        lax.fori_loop(0, CHUNK, row_body, 0, unroll=8)

    def group_body(g, _):
        acc = qT_v[pl.ds(g * 16, 16)]
        for c in range(1, 16):
            acc = acc + qT_v[pl.ds(c * QSTRIDE + g * 16, 16)]
        out_v[pl.ds(g * 16, 16)] = acc
        return 0

    lax.fori_loop(0, BPW // 16, group_body, 0, unroll=2)

    pltpu.sync_copy(out_v, out_hbm.at[pl.ds(base, BPW)])


def kernel(userids, itemids, user_memory, item_memory):
    return _mf_kernel(userids.astype(jnp.int32), itemids.astype(jnp.int32),
                      user_memory, item_memory)
